# Initial kernel scaffold; baseline (speedup 1.0000x reference)
import jax
import jax.numpy as jnp
from jax.experimental import pallas as pl


def kernel(z, weight):
    raise NotImplementedError("write your pallas kernel here")


# speedup vs baseline: 1.0908x; 1.0908x over previous
import functools

import jax
import jax.numpy as jnp
from jax import lax
from jax.experimental import pallas as pl
from jax.experimental.pallas import tpu as pltpu
from jax.experimental.pallas import tpu_sc as plsc

K = 8192
D = 256
N = 8192
BETA = 0.25
BN = 256


def _dist_argmin_body(f_ref, w_ref, x2_ref, w2_ref, idx_ref, mind_ref):
    f = f_ref[...]
    w = w_ref[...]
    mm = lax.dot_general(
        f, w, (((1,), (1,)), ((), ())),
        preferred_element_type=jnp.float32,
    )
    dist = (x2_ref[...] + w2_ref[...]) - 2.0 * mm
    lmin = jnp.min(dist, axis=1, keepdims=True)
    io = lax.broadcasted_iota(jnp.int32, dist.shape, 1)
    lidx = jnp.min(jnp.where(dist == lmin, io, K), axis=1, keepdims=True)
    idx_ref[...] = lidx
    mind_ref[...] = lmin


def _dist_argmin(flat, weight, x2, w2):
    grid = (N // BN,)
    return pl.pallas_call(
        _dist_argmin_body,
        grid=grid,
        in_specs=[
            pl.BlockSpec((BN, D), lambda i: (i, 0)),
            pl.BlockSpec((K, D), lambda i: (0, 0)),
            pl.BlockSpec((BN, 1), lambda i: (i, 0)),
            pl.BlockSpec((1, K), lambda i: (0, 0)),
        ],
        out_specs=[
            pl.BlockSpec((BN, 1), lambda i: (i, 0)),
            pl.BlockSpec((BN, 1), lambda i: (i, 0)),
        ],
        out_shape=[
            jax.ShapeDtypeStruct((N, 1), jnp.int32),
            jax.ShapeDtypeStruct((N, 1), jnp.float32),
        ],
    )(flat, weight, x2, w2)


def _sc_gather(weight, idx2d):
    info = plsc.get_sparse_core_info()
    nc, ns = info.num_cores, info.num_subcores
    nw = nc * ns
    b_per_w = N // nw
    chunks = b_per_w // 128

    mesh = plsc.VectorSubcoreMesh(core_axis_name="c", subcore_axis_name="s")

    @functools.partial(
        pl.kernel,
        out_type=jax.ShapeDtypeStruct((N, D), jnp.float32),
        mesh=mesh,
        scratch_types=[
            pltpu.VMEM((chunks, 128), jnp.int32),
            pltpu.VMEM((b_per_w, D), jnp.float32),
            pltpu.SemaphoreType.DMA,
        ],
    )
    def gather_kernel(table_hbm, idx_hbm, out_hbm, idx_v, rows_v, sem):
        wid = lax.axis_index("s") * nc + lax.axis_index("c")
        base = wid * b_per_w
        pltpu.sync_copy(idx_hbm.at[pl.ds(wid * chunks, chunks)], idx_v)
        for c in range(chunks):
            pltpu.async_copy(
                table_hbm.at[idx_v.at[c]],
                rows_v.at[pl.ds(c * 128, 128)],
                sem,
            ).wait()
        pltpu.sync_copy(rows_v, out_hbm.at[pl.ds(base, b_per_w)])

    return gather_kernel(weight, idx2d)


def kernel(z, weight):
    B, C, H, W = z.shape
    z_nhwc = jnp.transpose(z, (0, 2, 3, 1))
    flat = z_nhwc.reshape(-1, D)
    x2 = jnp.sum(flat ** 2, axis=1, keepdims=True)
    w2 = jnp.sum(weight ** 2, axis=1)

    idx2d, mind = _dist_argmin(flat, weight, x2, w2.reshape(1, K))
    idx = idx2d.reshape(N)

    z_q = _sc_gather(weight, idx.reshape(N // 128, 128))

    loss = BETA * (jnp.sum(mind) / (N * D))
    z_q_st = flat + (z_q - flat)
    z_q_out = jnp.transpose(z_q_st.reshape(B, H, W, C), (0, 3, 1, 2))
    return (z_q_out, idx, loss)


# speedup vs baseline: 1.1796x; 1.0813x over previous
import functools

import jax
import jax.numpy as jnp
from jax import lax
from jax.experimental import pallas as pl
from jax.experimental.pallas import tpu as pltpu
from jax.experimental.pallas import tpu_sc as plsc

K = 8192
D = 256
N = 8192
BETA = 0.25
BN = 512


def _dist_argmin_body(f_ref, w_ref, x2_ref, w2_ref, idx_ref, mind_ref):
    f = f_ref[...]
    w = w_ref[...]
    mm = lax.dot_general(
        f, w, (((1,), (1,)), ((), ())),
        preferred_element_type=jnp.float32,
    )
    dist = (x2_ref[...] + w2_ref[...]) - 2.0 * mm
    lmin = jnp.min(dist, axis=1, keepdims=True)
    io = lax.broadcasted_iota(jnp.int32, dist.shape, 1)
    lidx = jnp.min(jnp.where(dist == lmin, io, K), axis=1, keepdims=True)
    idx_ref[...] = lidx
    mind_ref[...] = lmin


def _dist_argmin(flat, weight, x2, w2):
    grid = (N // BN,)
    return pl.pallas_call(
        _dist_argmin_body,
        grid=grid,
        in_specs=[
            pl.BlockSpec((BN, D), lambda i: (i, 0)),
            pl.BlockSpec((K, D), lambda i: (0, 0)),
            pl.BlockSpec((BN, 1), lambda i: (i, 0)),
            pl.BlockSpec((1, K), lambda i: (0, 0)),
        ],
        out_specs=[
            pl.BlockSpec((BN, 1), lambda i: (i, 0)),
            pl.BlockSpec((BN, 1), lambda i: (i, 0)),
        ],
        out_shape=[
            jax.ShapeDtypeStruct((N, 1), jnp.int32),
            jax.ShapeDtypeStruct((N, 1), jnp.float32),
        ],
    )(flat, weight, x2, w2)


def _sc_gather(weight, idx2d):
    info = plsc.get_sparse_core_info()
    nc, ns = info.num_cores, info.num_subcores
    nw = nc * ns
    b_per_w = N // nw
    chunks = b_per_w // 128

    mesh = plsc.VectorSubcoreMesh(core_axis_name="c", subcore_axis_name="s")

    @functools.partial(
        pl.kernel,
        out_type=jax.ShapeDtypeStruct((N, D), jnp.float32),
        mesh=mesh,
        scratch_types=[
            pltpu.VMEM((chunks, 128), jnp.int32),
            pltpu.VMEM((b_per_w, D), jnp.float32),
            pltpu.SemaphoreType.DMA,
        ],
    )
    def gather_kernel(table_hbm, idx_hbm, out_hbm, idx_v, rows_v, sem):
        wid = lax.axis_index("s") * nc + lax.axis_index("c")
        base = wid * b_per_w
        pltpu.sync_copy(idx_hbm.at[pl.ds(wid * chunks, chunks)], idx_v)
        for c in range(chunks):
            pltpu.async_copy(
                table_hbm.at[idx_v.at[c]],
                rows_v.at[pl.ds(c * 128, 128)],
                sem,
            ).wait()
        pltpu.sync_copy(rows_v, out_hbm.at[pl.ds(base, b_per_w)])

    return gather_kernel(weight, idx2d)


def kernel(z, weight):
    B, C, H, W = z.shape
    z_nhwc = jnp.transpose(z, (0, 2, 3, 1))
    flat = z_nhwc.reshape(-1, D)
    x2 = jnp.sum(flat ** 2, axis=1, keepdims=True)
    w2 = jnp.sum(weight ** 2, axis=1)

    idx2d, mind = _dist_argmin(flat, weight, x2, w2.reshape(1, K))
    idx = idx2d.reshape(N)

    z_q = _sc_gather(weight, idx.reshape(N // 128, 128))

    loss = BETA * (jnp.sum(mind) / (N * D))
    z_q_st = flat + (z_q - flat)
    z_q_out = jnp.transpose(z_q_st.reshape(B, H, W, C), (0, 3, 1, 2))
    return (z_q_out, idx, loss)
